# Initial kernel scaffold; baseline (speedup 1.0000x reference)
import jax
import jax.numpy as jnp
from jax.experimental import pallas as pl


def kernel(x, dispatch_mask, combine_array, W1, b1, W2, b2):
    raise NotImplementedError("write your pallas kernel here")


# speedup vs baseline: 1.0624x; 1.0624x over previous
import jax
import jax.numpy as jnp
from jax.experimental import pallas as pl


def _erf(v):
    s = jnp.sign(v)
    av = jnp.abs(v)
    t = 1.0 / (1.0 + 0.3275911 * av)
    poly = t * (0.254829592 + t * (-0.284496736 + t * (1.421413741
           + t * (-1.453152027 + t * 1.061405429))))
    return s * (1.0 - poly * jnp.exp(-av * av))


def _gelu_exact(h):
    return 0.5 * h * (1.0 + _erf(h * 0.7071067811865476))


def _expert_step(dm_ref, cm_ref, x_ref, w1_ref, b1_ref, w2_ref, b2_ref, out_ref):
    e = pl.program_id(0)
    d = jax.lax.dot_general(
        dm_ref[0], x_ref[...],
        dimension_numbers=(((0,), (0,)), ((), ())),
        preferred_element_type=jnp.float32,
    )
    h = jnp.dot(d, w1_ref[0], preferred_element_type=jnp.float32) + b1_ref[0]
    h = _gelu_exact(h)
    y = jnp.dot(h, w2_ref[0], preferred_element_type=jnp.float32) + b2_ref[0]
    contrib = jnp.dot(cm_ref[0], y, preferred_element_type=jnp.float32)

    @pl.when(e == 0)
    def _init():
        out_ref[...] = contrib

    @pl.when(e != 0)
    def _acc():
        out_ref[...] += contrib


def kernel(x, dispatch_mask, combine_array, W1, b1, W2, b2):
    B, S, D = x.shape
    _, _, E, C = dispatch_mask.shape
    HE = W1.shape[2]

    xs = x[0]
    dm = dispatch_mask[0].transpose(1, 0, 2)
    cm = combine_array[0].transpose(1, 0, 2)
    b1r = b1.reshape(E, 1, HE)
    b2r = b2.reshape(E, 1, D)

    out = pl.pallas_call(
        _expert_step,
        grid=(E,),
        in_specs=[
            pl.BlockSpec((1, S, C), lambda e: (e, 0, 0)),
            pl.BlockSpec((1, S, C), lambda e: (e, 0, 0)),
            pl.BlockSpec((S, D), lambda e: (0, 0)),
            pl.BlockSpec((1, D, HE), lambda e: (e, 0, 0)),
            pl.BlockSpec((1, 1, HE), lambda e: (e, 0, 0)),
            pl.BlockSpec((1, HE, D), lambda e: (e, 0, 0)),
            pl.BlockSpec((1, 1, D), lambda e: (e, 0, 0)),
        ],
        out_specs=pl.BlockSpec((S, D), lambda e: (0, 0)),
        out_shape=jax.ShapeDtypeStruct((S, D), jnp.float32),
    )(dm, cm, xs, W1, b1r, W2, b2r)
    return out[None]
